# baseline (speedup 1.0000x reference)
import functools

import jax
import jax.numpy as jnp
from jax import lax
from jax.experimental import pallas as pl
from jax.experimental.pallas import tpu as pltpu
from jax.experimental.pallas import tpu_sc as plsc

NUM_ROWS = 1000000
BATCH = 16384
HIST = 200
DIM = 64
NTOT = BATCH * HIST
NW = 32
B_PER_W = NTOT // NW
CHUNK = 512
NCHUNK = B_PER_W // CHUNK
NBUF = 2
NIDX = 4
NOUTER = NCHUNK // NIDX
SPLIT = 4
SUB = CHUNK // SPLIT

_mesh = plsc.VectorSubcoreMesh(core_axis_name="c", subcore_axis_name="s")


@functools.partial(
    pl.kernel,
    mesh=_mesh,
    out_type=jax.ShapeDtypeStruct((NTOT, DIM), jnp.float32),
    scratch_types=(
        [pltpu.VMEM((CHUNK,), jnp.int32) for _ in range(NIDX)]
        + [pltpu.VMEM((CHUNK, DIM), jnp.float32) for _ in range(NBUF)]
        + [pltpu.SemaphoreType.DMA for _ in range(NIDX)]
        + [pltpu.SemaphoreType.DMA for _ in range(NBUF)]
        + [pltpu.SemaphoreType.DMA for _ in range(NBUF)]
    ),
    compiler_params=pltpu.CompilerParams(use_tc_tiling_on_sc=False),
)
def _gather(idx_hbm, table_hbm, out_hbm, *scratch):
    idx_v = scratch[0:NIDX]
    rows_v = scratch[NIDX:NIDX + NBUF]
    sem_i = scratch[NIDX + NBUF:2 * NIDX + NBUF]
    sem_g = scratch[2 * NIDX + NBUF:2 * NIDX + 2 * NBUF]
    sem_s = scratch[2 * NIDX + 2 * NBUF:2 * NIDX + 3 * NBUF]
    wid = lax.axis_index("s") * 2 + lax.axis_index("c")
    base = wid * B_PER_W

    def load_idx(q, off):
        pltpu.async_copy(idx_hbm.at[pl.ds(off, CHUNK)], idx_v[q], sem_i[q])

    def wait_idx(q, off):
        pltpu.make_async_copy(
            idx_hbm.at[pl.ds(off, CHUNK)], idx_v[q], sem_i[q]).wait()

    def issue_gather(b, q):
        for k in range(SPLIT):
            pltpu.async_copy(
                table_hbm.at[idx_v[q].at[pl.ds(k * SUB, SUB)]],
                rows_v[b].at[pl.ds(k * SUB, SUB)], sem_g[b])

    def wait_gather(b, q):
        for k in range(SPLIT):
            pltpu.make_async_copy(
                table_hbm.at[idx_v[q].at[pl.ds(k * SUB, SUB)]],
                rows_v[b].at[pl.ds(k * SUB, SUB)], sem_g[b]).wait()

    def store(b, off):
        return rows_v[b], out_hbm.at[pl.ds(off, CHUNK)]

    for q in range(NIDX):
        load_idx(q, base + q * CHUNK)
    for b in range(NBUF):
        wait_idx(b, base + b * CHUNK)
        issue_gather(b, b)

    def body(g, carry):
        for j in range(NIDX):
            b = j % NBUF
            q = j
            i = g * NIDX + j
            off = base + i * CHUNK
            wait_gather(b, q)
            src, dst = store(b, off)
            pltpu.async_copy(src, dst, sem_s[b])

            @pl.when(i + NIDX < NCHUNK)
            def _():
                load_idx(q, off + NIDX * CHUNK)

            @pl.when(i + NBUF < NCHUNK)
            def _():
                src2, dst2 = store(b, off)
                pltpu.make_async_copy(src2, dst2, sem_s[b]).wait()
                off2 = off + NBUF * CHUNK
                q2 = (j + NBUF) % NIDX
                wait_idx(q2, off2)
                issue_gather(b, q2)

        return carry

    lax.fori_loop(0, NOUTER, body, 0)

    for b in range(NBUF):
        off = base + (NCHUNK - NBUF + b) * CHUNK
        src, dst = store(b, off)
        pltpu.make_async_copy(src, dst, sem_s[b]).wait()


def kernel(prompt_ids, weight):
    flat = prompt_ids.reshape(NTOT).astype(jnp.int32)
    out = _gather(flat, weight)
    return out.reshape(BATCH, HIST, DIM)


# speedup vs baseline: 1.0019x; 1.0019x over previous
import functools

import jax
import jax.numpy as jnp
from jax import lax
from jax.experimental import pallas as pl
from jax.experimental.pallas import tpu as pltpu
from jax.experimental.pallas import tpu_sc as plsc

NUM_ROWS = 1000000
BATCH = 16384
HIST = 200
DIM = 64
NTOT = BATCH * HIST
NW = 32
B_PER_W = NTOT // NW
CHUNK = 512
NCHUNK = B_PER_W // CHUNK
NBUF = 2
NIDX = 4
NOUTER = NCHUNK // NIDX
SPLIT = 4
SUB = CHUNK // SPLIT

_mesh = plsc.VectorSubcoreMesh(core_axis_name="c", subcore_axis_name="s")


@functools.partial(
    pl.kernel,
    mesh=_mesh,
    out_type=jax.ShapeDtypeStruct((NTOT, DIM), jnp.float32),
    scratch_types=(
        [pltpu.VMEM((CHUNK,), jnp.int32) for _ in range(NIDX)]
        + [pltpu.VMEM((CHUNK, DIM), jnp.float32) for _ in range(NBUF)]
        + [pltpu.SemaphoreType.DMA for _ in range(NIDX)]
        + [pltpu.SemaphoreType.DMA for _ in range(NBUF)]
        + [pltpu.SemaphoreType.DMA for _ in range(NBUF)]
    ),
    compiler_params=pltpu.CompilerParams(
        use_tc_tiling_on_sc=False, skip_device_barrier=True),
)
def _gather(idx_hbm, table_hbm, out_hbm, *scratch):
    idx_v = scratch[0:NIDX]
    rows_v = scratch[NIDX:NIDX + NBUF]
    sem_i = scratch[NIDX + NBUF:2 * NIDX + NBUF]
    sem_g = scratch[2 * NIDX + NBUF:2 * NIDX + 2 * NBUF]
    sem_s = scratch[2 * NIDX + 2 * NBUF:2 * NIDX + 3 * NBUF]
    wid = lax.axis_index("s") * 2 + lax.axis_index("c")
    base = wid * B_PER_W

    def load_idx(q, off):
        pltpu.async_copy(idx_hbm.at[pl.ds(off, CHUNK)], idx_v[q], sem_i[q])

    def wait_idx(q, off):
        pltpu.make_async_copy(
            idx_hbm.at[pl.ds(off, CHUNK)], idx_v[q], sem_i[q]).wait()

    def issue_gather(b, q):
        for k in range(SPLIT):
            pltpu.async_copy(
                table_hbm.at[idx_v[q].at[pl.ds(k * SUB, SUB)]],
                rows_v[b].at[pl.ds(k * SUB, SUB)], sem_g[b])

    def wait_gather(b, q):
        for k in range(SPLIT):
            pltpu.make_async_copy(
                table_hbm.at[idx_v[q].at[pl.ds(k * SUB, SUB)]],
                rows_v[b].at[pl.ds(k * SUB, SUB)], sem_g[b]).wait()

    def store(b, off):
        return rows_v[b], out_hbm.at[pl.ds(off, CHUNK)]

    for q in range(NIDX):
        load_idx(q, base + q * CHUNK)
    for b in range(NBUF):
        wait_idx(b, base + b * CHUNK)
        issue_gather(b, b)

    def body(g, carry):
        for j in range(NIDX):
            b = j % NBUF
            q = j
            i = g * NIDX + j
            off = base + i * CHUNK
            wait_gather(b, q)
            src, dst = store(b, off)
            pltpu.async_copy(src, dst, sem_s[b])

            @pl.when(i + NIDX < NCHUNK)
            def _():
                load_idx(q, off + NIDX * CHUNK)

            @pl.when(i + NBUF < NCHUNK)
            def _():
                src2, dst2 = store(b, off)
                pltpu.make_async_copy(src2, dst2, sem_s[b]).wait()
                off2 = off + NBUF * CHUNK
                q2 = (j + NBUF) % NIDX
                wait_idx(q2, off2)
                issue_gather(b, q2)

        return carry

    lax.fori_loop(0, NOUTER, body, 0)

    for b in range(NBUF):
        off = base + (NCHUNK - NBUF + b) * CHUNK
        src, dst = store(b, off)
        pltpu.make_async_copy(src, dst, sem_s[b]).wait()


def kernel(prompt_ids, weight):
    flat = prompt_ids.reshape(NTOT).astype(jnp.int32)
    out = _gather(flat, weight)
    return out.reshape(BATCH, HIST, DIM)
